# baseline (speedup 1.0000x reference)
import functools

import jax
import jax.numpy as jnp
from jax.experimental import pallas as pl
from jax.experimental.pallas import tpu as pltpu
from jax.experimental.pallas import tpu_sc as plsc

_B = 4
_D = 2048

_mesh = plsc.ScalarSubcoreMesh(axis_name="c", num_cores=1)


@functools.partial(
    pl.kernel,
    out_type=jax.ShapeDtypeStruct((_B, 2, _D), jnp.float32),
    mesh=_mesh,
    scratch_types=[pltpu.SemaphoreType.DMA],
)
def _first_last_pool(h_hbm, out_hbm, sem):
    copies = [
        pltpu.async_copy(h_hbm.at[:, 0, :], out_hbm.at[:, slot, :], sem)
        for slot in range(2)
    ]
    for c in copies:
        c.wait()


def kernel(hidden_state):
    return _first_last_pool(hidden_state)
